# baseline (speedup 1.0000x reference)
import numpy as np
import jax
import jax.numpy as jnp
from jax import lax
from jax.experimental import pallas as pl
from jax.experimental.pallas import tpu as pltpu

_G = 23
_SZ = _G * _G
_ORG = _G + 1
_BUF = _SZ + 2 * _ORG


def _lrelu_mask(y, ones_col):
    y = jnp.where(y > 0, y, 0.2 * y)
    if ones_col:
        y = jnp.concatenate([y, jnp.ones((y.shape[0], 1), y.dtype)], axis=1)
    r = lax.broadcasted_iota(jnp.int32, y.shape, 0)
    valid = (r < 21 * _G) & (r % _G < 21)
    return jnp.where(valid, y, 0.0)


def _conv9(z_ref, w_ref):
    acc = None
    for kh in range(3):
        for kw in range(3):
            start = _ORG + (kh - 1) * _G + (kw - 1)
            xs = z_ref[start:start + _SZ, :]
            p = lax.dot_general(xs, w_ref[kh * 3 + kw], (((1,), (0,)), ((), ())),
                                preferred_element_type=jnp.float32)
            acc = p if acc is None else acc + p
    return acc


def _encoder_kernel(x_ref, w1_ref, w2_ref, w3_ref, w4_ref, out_ref,
                    z2, z3, z4):
    @pl.when(pl.program_id(0) == 0)
    def _init():
        z2[...] = jnp.zeros_like(z2)
        z3[...] = jnp.zeros_like(z3)
        z4[...] = jnp.zeros_like(z4)

    for img in range(x_ref.shape[0]):
        for p2 in range(2):
            for q2 in range(2):
                m = None
                for p1 in range(2):
                    for q1 in range(2):
                        ph = ((p1 * 2 + q1) * 2 + p2) * 2 + q2
                        y = lax.dot_general(x_ref[img], w1_ref[ph],
                                            (((1,), (0,)), ((), ())),
                                            preferred_element_type=jnp.float32)
                        m = y if m is None else jnp.maximum(m, y)
                m = _lrelu_mask(m, True)
                z2[p2 * 2 + q2, _ORG:_ORG + _SZ, :] = m.astype(jnp.bfloat16)

        pooled = None
        for p in range(2):
            for q in range(2):
                acc = None
                for kh in range(3):
                    for kw in range(3):
                        u = p + kh - 1
                        v = q + kw - 1
                        pp, aa = u % 2, (u - u % 2) // 2
                        qq, bb = v % 2, (v - v % 2) // 2
                        start = _ORG + aa * _G + bb
                        xs = z2[pp * 2 + qq, start:start + _SZ, :]
                        t = lax.dot_general(xs, w2_ref[kh * 3 + kw],
                                            (((1,), (0,)), ((), ())),
                                            preferred_element_type=jnp.float32)
                        acc = t if acc is None else acc + t
                pooled = acc if pooled is None else jnp.maximum(pooled, acc)
        y = _lrelu_mask(pooled, True)
        z3[_ORG:_ORG + _SZ, :] = y.astype(jnp.bfloat16)

        y = _lrelu_mask(_conv9(z3, w3_ref), True)
        z4[_ORG:_ORG + _SZ, :] = y.astype(jnp.bfloat16)
        y = _lrelu_mask(_conv9(z4, w4_ref), False)

        n = jnp.sqrt(jnp.sum(y * y, axis=1, keepdims=True))
        out_ref[img] = (y / jnp.maximum(n, 1e-12)).astype(jnp.bfloat16)


def _vote_kernel(q_ref, s_ref, out_ref):
    way = s_ref.shape[1]
    ninf = jnp.bfloat16(-jnp.inf)
    for img in range(q_ref.shape[0]):
        qn = q_ref[img]
        accs = []
        for c in range(way):
            sn = s_ref[0, c]
            sim = lax.dot_general(qn, sn, (((1,), (1,)), ((), ())),
                                  preferred_element_type=jnp.float32
                                  ).astype(jnp.bfloat16)
            m1 = jnp.max(sim, axis=1, keepdims=True)
            eq1 = sim == m1
            one = jnp.bfloat16(1.0)
            zero = jnp.bfloat16(0.0)
            n1 = jnp.sum(jnp.where(eq1, one, zero), axis=1, keepdims=True)
            s2 = jnp.where(eq1, ninf, sim)
            m2 = jnp.max(s2, axis=1, keepdims=True)
            eq2 = s2 == m2
            n2 = jnp.sum(jnp.where(eq2, one, zero), axis=1, keepdims=True)
            m3 = jnp.max(jnp.where(eq2, ninf, s2), axis=1, keepdims=True)
            t2 = jnp.where(n1 >= 2, m1, m2)
            t3 = jnp.where(n1 >= 3, m1,
                           jnp.where((n1 == 2) | (n2 >= 2), m2, m3))
            row_tot = (m1.astype(jnp.float32) + t2.astype(jnp.float32)
                       + t3.astype(jnp.float32))
            accs.append(jnp.sum(row_tot, axis=0, keepdims=True))
        out_ref[img] = jnp.concatenate(accs, axis=1)


def kernel(query, support, W1, g1, b1, rm1, rv1, W2, g2, b2, rm2, rv2,
           W3, g3, b3, rm3, rv3, W4, g4, b4, rm4, rv4):
    B, NQ, C, H, W = query.shape
    _, Way, Shot, _, _, _ = support.shape
    N = B * NQ + B * Way * Shot
    D = 64

    x = jnp.concatenate([query.reshape(-1, C, H, W),
                         support.reshape(-1, C, H, W)], axis=0)
    patches = lax.conv_general_dilated_patches(
        x, (7, 7), (4, 4), ((1, 13), (1, 13)),
        dimension_numbers=("NCHW", "OIHW", "NHWC"))
    x1col = patches.reshape(N, _SZ, 49 * C)
    x1col = jnp.concatenate(
        [x1col, jnp.ones((N, _SZ, 1), x1col.dtype)], axis=2).astype(jnp.bfloat16)

    sc, bi = [], []
    for g, b, rm, rv in ((g1, b1, rm1, rv1), (g2, b2, rm2, rv2),
                         (g3, b3, rm3, rv3), (g4, b4, rm4, rv4)):
        s = g * lax.rsqrt(rv + 1e-5)
        sc.append(s)
        bi.append(b - rm * s)

    w1e = jnp.zeros((16, 49 * C + 1, D), W1.dtype)
    for p1 in range(2):
        for q1 in range(2):
            for p2 in range(2):
                for q2 in range(2):
                    ph = ((p1 * 2 + q1) * 2 + p2) * 2 + q2
                    for kh in range(3):
                        for kw in range(3):
                            u = 2 * p2 + p1 + kh
                            v = 2 * q2 + q1 + kw
                            f = np.arange(C) * 49 + u * 7 + v
                            w1e = w1e.at[ph, f, :].set(
                                W1[:, :, kh, kw].T * sc[0][None, :])
                    w1e = w1e.at[ph, 49 * C, :].set(bi[0])

    def _fold(Wl, s, b):
        wr = Wl.transpose(2, 3, 1, 0).reshape(9, D, D) * s[None, None, :]
        wf = jnp.zeros((9, D + 1, D), Wl.dtype)
        wf = wf.at[:, :D, :].set(wr)
        return wf.at[4, D, :].set(b)

    w2r = _fold(W2, sc[1], bi[1])
    w3r = _fold(W3, sc[2], bi[2])
    w4r = _fold(W4, sc[3], bi[3])

    EB = 4
    feats = pl.pallas_call(
        _encoder_kernel,
        grid=(N // EB,),
        in_specs=[
            pl.BlockSpec((EB, _SZ, 49 * C + 1), lambda i: (i, 0, 0)),
            pl.BlockSpec((16, 49 * C + 1, D), lambda i: (0, 0, 0)),
            pl.BlockSpec((9, D + 1, D), lambda i: (0, 0, 0)),
            pl.BlockSpec((9, D + 1, D), lambda i: (0, 0, 0)),
            pl.BlockSpec((9, D + 1, D), lambda i: (0, 0, 0)),
        ],
        out_specs=pl.BlockSpec((EB, _SZ, D), lambda i: (i, 0, 0)),
        out_shape=jax.ShapeDtypeStruct((N, _SZ, D), jnp.bfloat16),
        scratch_shapes=[
            pltpu.VMEM((4, _BUF, D + 1), jnp.bfloat16),
            pltpu.VMEM((_BUF, D + 1), jnp.bfloat16),
            pltpu.VMEM((_BUF, D + 1), jnp.bfloat16),
        ],
    )(x1col, w1e.astype(jnp.bfloat16), w2r.astype(jnp.bfloat16),
      w3r.astype(jnp.bfloat16), w4r.astype(jnp.bfloat16))

    cols441 = np.array([a * _G + b for a in range(21) for b in range(21)],
                       dtype=np.int32)
    featsb = feats[:, cols441, :]

    L = 21 * 21
    M = Shot * L
    q_local = featsb[:B * NQ]
    s_local = featsb[B * NQ:].reshape(B, Way, M, D)

    QB = 5
    scores = pl.pallas_call(
        _vote_kernel,
        grid=(B * NQ // QB,),
        in_specs=[
            pl.BlockSpec((QB, L, D), lambda i: (i, 0, 0)),
            pl.BlockSpec((1, Way, M, D), lambda i: (i * QB // NQ, 0, 0, 0)),
        ],
        out_specs=pl.BlockSpec((QB, 1, Way), lambda i: (i, 0, 0)),
        out_shape=jax.ShapeDtypeStruct((B * NQ, 1, Way), jnp.float32),
    )(q_local, s_local)
    return scores.reshape(B * NQ, Way)


# speedup vs baseline: 1.0898x; 1.0898x over previous
import numpy as np
import jax
import jax.numpy as jnp
from jax import lax
from jax.experimental import pallas as pl
from jax.experimental.pallas import tpu as pltpu

_G = 23
_SZ = _G * _G
_ORG = _G + 1
_BUF = _SZ + 2 * _ORG


def _lrelu_mask(y, ones_col):
    y = jnp.where(y > 0, y, 0.2 * y)
    if ones_col:
        y = jnp.concatenate([y, jnp.ones((y.shape[0], 1), y.dtype)], axis=1)
    r = lax.broadcasted_iota(jnp.int32, y.shape, 0)
    valid = (r < 21 * _G) & (r % _G < 21)
    return jnp.where(valid, y, 0.0)


def _conv9(z_ref, w_ref):
    acc = None
    for kh in range(3):
        for kw in range(3):
            start = _ORG + (kh - 1) * _G + (kw - 1)
            xs = z_ref[start:start + _SZ, :]
            p = lax.dot_general(xs, w_ref[kh * 3 + kw], (((1,), (0,)), ((), ())),
                                preferred_element_type=jnp.float32)
            acc = p if acc is None else acc + p
    return acc


def _encoder_kernel(x_ref, w1_ref, w2_ref, w3_ref, w4_ref, out_ref,
                    z2, z3, z4):
    @pl.when(pl.program_id(0) == 0)
    def _init():
        z2[...] = jnp.zeros_like(z2)
        z3[...] = jnp.zeros_like(z3)
        z4[...] = jnp.zeros_like(z4)

    for img in range(x_ref.shape[0]):
        for p2 in range(2):
            for q2 in range(2):
                m = None
                for p1 in range(2):
                    for q1 in range(2):
                        ph = ((p1 * 2 + q1) * 2 + p2) * 2 + q2
                        y = lax.dot_general(x_ref[img], w1_ref[ph],
                                            (((1,), (0,)), ((), ())),
                                            preferred_element_type=jnp.float32)
                        m = y if m is None else jnp.maximum(m, y)
                m = _lrelu_mask(m, True)
                z2[p2 * 2 + q2, _ORG:_ORG + _SZ, :] = m.astype(jnp.bfloat16)

        pooled = None
        for p in range(2):
            for q in range(2):
                acc = None
                for kh in range(3):
                    for kw in range(3):
                        u = p + kh - 1
                        v = q + kw - 1
                        pp, aa = u % 2, (u - u % 2) // 2
                        qq, bb = v % 2, (v - v % 2) // 2
                        start = _ORG + aa * _G + bb
                        xs = z2[pp * 2 + qq, start:start + _SZ, :]
                        t = lax.dot_general(xs, w2_ref[kh * 3 + kw],
                                            (((1,), (0,)), ((), ())),
                                            preferred_element_type=jnp.float32)
                        acc = t if acc is None else acc + t
                pooled = acc if pooled is None else jnp.maximum(pooled, acc)
        y = _lrelu_mask(pooled, True)
        z3[_ORG:_ORG + _SZ, :] = y.astype(jnp.bfloat16)

        y = _lrelu_mask(_conv9(z3, w3_ref), True)
        z4[_ORG:_ORG + _SZ, :] = y.astype(jnp.bfloat16)
        y = _lrelu_mask(_conv9(z4, w4_ref), False)

        n = jnp.sqrt(jnp.sum(y * y, axis=1, keepdims=True))
        out_ref[img] = (y / jnp.maximum(n, 1e-12)).astype(jnp.bfloat16)


def _vote_kernel(q_ref, s_ref, out_ref):
    way = s_ref.shape[1]
    ninf = jnp.bfloat16(-jnp.inf)
    for img in range(q_ref.shape[0]):
        qn = q_ref[img]
        accs = []
        for c in range(way):
            sn = s_ref[0, c]
            sim = lax.dot_general(qn, sn, (((1,), (1,)), ((), ())),
                                  preferred_element_type=jnp.float32
                                  ).astype(jnp.bfloat16)
            m1 = jnp.max(sim, axis=1, keepdims=True)
            eq1 = sim == m1
            one = jnp.bfloat16(1.0)
            zero = jnp.bfloat16(0.0)
            n1 = jnp.sum(jnp.where(eq1, one, zero), axis=1, keepdims=True)
            s2 = jnp.where(eq1, ninf, sim)
            m2 = jnp.max(s2, axis=1, keepdims=True)
            eq2 = s2 == m2
            n2 = jnp.sum(jnp.where(eq2, one, zero), axis=1, keepdims=True)
            m3 = jnp.max(jnp.where(eq2, ninf, s2), axis=1, keepdims=True)
            t2 = jnp.where(n1 >= 2, m1, m2)
            t3 = jnp.where(n1 >= 3, m1,
                           jnp.where((n1 == 2) | (n2 >= 2), m2, m3))
            row_tot = (m1.astype(jnp.float32) + t2.astype(jnp.float32)
                       + t3.astype(jnp.float32))
            accs.append(jnp.sum(row_tot, axis=0, keepdims=True))
        out_ref[img] = jnp.concatenate(accs, axis=1)


def kernel(query, support, W1, g1, b1, rm1, rv1, W2, g2, b2, rm2, rv2,
           W3, g3, b3, rm3, rv3, W4, g4, b4, rm4, rv4):
    B, NQ, C, H, W = query.shape
    _, Way, Shot, _, _, _ = support.shape
    N = B * NQ + B * Way * Shot
    D = 64

    x = jnp.concatenate([query.reshape(-1, C, H, W),
                         support.reshape(-1, C, H, W)], axis=0)
    patches = lax.conv_general_dilated_patches(
        x, (7, 7), (4, 4), ((1, 13), (1, 13)),
        dimension_numbers=("NCHW", "OIHW", "NHWC"))
    x1col = patches.reshape(N, _SZ, 49 * C)
    x1col = jnp.concatenate(
        [x1col, jnp.ones((N, _SZ, 1), x1col.dtype)], axis=2).astype(jnp.bfloat16)

    sc, bi = [], []
    for g, b, rm, rv in ((g1, b1, rm1, rv1), (g2, b2, rm2, rv2),
                         (g3, b3, rm3, rv3), (g4, b4, rm4, rv4)):
        s = g * lax.rsqrt(rv + 1e-5)
        sc.append(s)
        bi.append(b - rm * s)

    w1e = jnp.zeros((16, 49 * C + 1, D), W1.dtype)
    for p1 in range(2):
        for q1 in range(2):
            for p2 in range(2):
                for q2 in range(2):
                    ph = ((p1 * 2 + q1) * 2 + p2) * 2 + q2
                    for kh in range(3):
                        for kw in range(3):
                            u = 2 * p2 + p1 + kh
                            v = 2 * q2 + q1 + kw
                            f = np.arange(C) * 49 + u * 7 + v
                            w1e = w1e.at[ph, f, :].set(
                                W1[:, :, kh, kw].T * sc[0][None, :])
                    w1e = w1e.at[ph, 49 * C, :].set(bi[0])

    def _fold(Wl, s, b):
        wr = Wl.transpose(2, 3, 1, 0).reshape(9, D, D) * s[None, None, :]
        wf = jnp.zeros((9, D + 1, D), Wl.dtype)
        wf = wf.at[:, :D, :].set(wr)
        return wf.at[4, D, :].set(b)

    w2r = _fold(W2, sc[1], bi[1])
    w3r = _fold(W3, sc[2], bi[2])
    w4r = _fold(W4, sc[3], bi[3])

    EB = 4
    feats = pl.pallas_call(
        _encoder_kernel,
        grid=(N // EB,),
        in_specs=[
            pl.BlockSpec((EB, _SZ, 49 * C + 1), lambda i: (i, 0, 0)),
            pl.BlockSpec((16, 49 * C + 1, D), lambda i: (0, 0, 0)),
            pl.BlockSpec((9, D + 1, D), lambda i: (0, 0, 0)),
            pl.BlockSpec((9, D + 1, D), lambda i: (0, 0, 0)),
            pl.BlockSpec((9, D + 1, D), lambda i: (0, 0, 0)),
        ],
        out_specs=pl.BlockSpec((EB, _SZ, D), lambda i: (i, 0, 0)),
        out_shape=jax.ShapeDtypeStruct((N, _SZ, D), jnp.bfloat16),
        scratch_shapes=[
            pltpu.VMEM((4, _BUF, D + 1), jnp.bfloat16),
            pltpu.VMEM((_BUF, D + 1), jnp.bfloat16),
            pltpu.VMEM((_BUF, D + 1), jnp.bfloat16),
        ],
    )(x1col, w1e.astype(jnp.bfloat16), w2r.astype(jnp.bfloat16),
      w3r.astype(jnp.bfloat16), w4r.astype(jnp.bfloat16))

    cols441 = np.array([a * _G + b for a in range(21) for b in range(21)],
                       dtype=np.int32)
    featsb = feats[:, cols441, :]

    L = 21 * 21
    M = Shot * L
    q_local = featsb[:B * NQ]
    s_local = featsb[B * NQ:].reshape(B, Way, M, D)

    QB = 3
    scores = pl.pallas_call(
        _vote_kernel,
        grid=(B * NQ // QB,),
        in_specs=[
            pl.BlockSpec((QB, L, D), lambda i: (i, 0, 0)),
            pl.BlockSpec((1, Way, M, D), lambda i: (i * QB // NQ, 0, 0, 0)),
        ],
        out_specs=pl.BlockSpec((QB, 1, Way), lambda i: (i, 0, 0)),
        out_shape=jax.ShapeDtypeStruct((B * NQ, 1, Way), jnp.float32),
    )(q_local, s_local)
    return scores.reshape(B * NQ, Way)
